# baseline (speedup 1.0000x reference)
import functools

import jax
import jax.numpy as jnp
from jax import lax
from jax.experimental import pallas as pl
from jax.experimental.pallas import tpu as pltpu
from jax.experimental.pallas import tpu_sc as plsc

N_USERS = 20000
N_ENT = 80000
N = N_USERS + N_ENT
D = 64
NNZ = 1000000
BATCH = 4096

NC = 2
NS = 16
NW = NC * NS
EPB = 128
PSTEPS = 32
NP = 8
S = PSTEPS * NP
NNZ_PAD = NW * S * EPB
SUB_BLK = 6256
N_PAD = NS * SUB_BLK
ACC_ROWS = N_PAD + 8
ZROWS = 368

_f32 = jnp.float32
_i32 = jnp.int32


def _make_spmm(nchunk):
    mesh = plsc.VectorSubcoreMesh(
        core_axis_name="c", subcore_axis_name="s", num_cores=NC,
        num_subcores=NS)

    scratch = [
        pltpu.VMEM((PSTEPS, EPB), _i32),
        pltpu.VMEM((PSTEPS, EPB), _i32),
        pltpu.VMEM((PSTEPS, EPB), _f32),
        pltpu.VMEM((EPB, 16), _f32),
        pltpu.VMEM((EPB, 16), _f32),
        pltpu.VMEM((EPB, 16), _f32),
        pltpu.VMEM((EPB, 16), _f32),
        pltpu.VMEM((ZROWS, 16), _f32),
        pltpu.SemaphoreType.DMA,
        pltpu.SemaphoreType.DMA,
        pltpu.SemaphoreType.DMA,
        pltpu.SemaphoreType.DMA,
        pltpu.SemaphoreType.DMA,
        pltpu.SemaphoreType.DMA,
        pltpu.SemaphoreType.DMA,
        pltpu.SemaphoreType.DMA,
        pltpu.MemorySpace.VMEM_SHARED((ACC_ROWS, 16), _f32),
    ]

    def _body(rows_h, cols_h, vals_h, xs, out_h,
              rows_v, cols_v, vals_v, g0, g1, g2, g3, zbuf,
              gs0, gs1, gs2, gs3, ss0, ss1, ss2, ss3, acc):
        c = lax.axis_index("c")
        s = lax.axis_index("s")
        w = s * NC + c

        zero16 = jnp.zeros((16,), _f32)

        def _zfill(i, carry):
            zbuf[i, :] = zero16
            return carry

        lax.fori_loop(0, ZROWS, _zfill, 0)

        base = s * SUB_BLK
        bufs = (g0, g1, g2, g3)
        gsems = (gs0, gs1, gs2, gs3)
        ssems = (ss0, ss1, ss2, ss3)
        nbuf = 4
        npre = 2

        for k in range(nchunk):
            xk = xs[k]

            def _zero_acc(i, carry):
                pltpu.sync_copy(zbuf, acc.at[pl.ds(base + i * ZROWS, ZROWS)])
                return carry

            lax.fori_loop(0, SUB_BLK // ZROWS, _zero_acc, 0)
            plsc.subcore_barrier()

            def _subround(p, carry0):
                poff = pl.multiple_of(p * PSTEPS, PSTEPS)
                pltpu.sync_copy(rows_h.at[w, pl.ds(poff, PSTEPS)], rows_v)
                pltpu.sync_copy(cols_h.at[w, pl.ds(poff, PSTEPS)], cols_v)
                pltpu.sync_copy(vals_h.at[w, pl.ds(poff, PSTEPS)], vals_v)

                for b in range(npre):
                    pltpu.async_copy(xk.at[cols_v.at[b]], bufs[b], gsems[b])

                def _step(j0, carry):
                    for b in range(nbuf):
                        j = j0 * nbuf + b
                        bp = (b + npre) % nbuf
                        gbuf = bufs[b]
                        pltpu.make_async_copy(
                            xk.at[cols_v.at[j]], gbuf, gsems[b]).wait()

                        def _scale(g, c2):
                            goff = pl.multiple_of(g * 16, 16)
                            vgrp = vals_v[j, pl.ds(goff, 16)]
                            for e in range(16):
                                v16 = vgrp.at[jnp.full((16,), e, _i32)].get(
                                    mode="promise_in_bounds")
                                gbuf[goff + e, :] = gbuf[goff + e, :] * v16
                            return c2

                        lax.fori_loop(0, EPB // 16, _scale, 0)
                        pltpu.async_copy(
                            gbuf, acc.at[rows_v.at[j]], ssems[b], add=True)

                        @pl.when(j >= npre)
                        def _drain_prev():
                            jq = j - npre
                            pltpu.make_async_copy(
                                bufs[bp], acc.at[rows_v.at[jq]],
                                ssems[bp]).wait()

                        @pl.when(j + npre < PSTEPS)
                        def _start_next():
                            pltpu.async_copy(
                                xk.at[cols_v.at[j + npre]], bufs[bp],
                                gsems[bp])
                    return carry

                lax.fori_loop(0, PSTEPS // nbuf, _step, 0)
                for b in range(npre):
                    jq = PSTEPS - npre + b
                    pltpu.make_async_copy(
                        bufs[jq % nbuf], acc.at[rows_v.at[jq]],
                        ssems[jq % nbuf]).wait()
                return carry0

            lax.fori_loop(0, NP, _subround, 0)

            plsc.subcore_barrier()
            pltpu.sync_copy(
                acc.at[pl.ds(base, SUB_BLK)],
                out_h.at[c, k, pl.ds(base, SUB_BLK)])
            plsc.subcore_barrier()

    @functools.partial(
        pl.kernel, mesh=mesh,
        out_type=jax.ShapeDtypeStruct((NC, nchunk, N_PAD, 16), _f32),
        scratch_types=scratch,
        compiler_params=pltpu.CompilerParams(use_tc_tiling_on_sc=False),
    )
    def spmm(rows_h, cols_h, vals_h, *rest):
        xs = rest[:nchunk]
        out_h = rest[nchunk]
        _body(rows_h, cols_h, vals_h, xs, out_h, *rest[nchunk + 1:])

    return spmm


_SPMM = {d: _make_spmm(d // 16) for d in (64, 32, 16)}


def _spmm_sc(rows3, cols3, vals3, x):
    d = x.shape[1]
    nchunk = d // 16
    chunks = [x[:, 16 * k:16 * (k + 1)] for k in range(nchunk)]
    out = _SPMM[d](rows3, cols3, vals3, *chunks)
    part = out[0] + out[1]
    return jnp.moveaxis(part, 0, 1)[:N].reshape(N, d)


def _layer_body(x_ref, sd_ref, w1_ref, b1_ref, w2_ref, b2_ref,
                ego_ref, nrm_ref):
    x = x_ref[...]
    side = sd_ref[...]
    a = jnp.dot(x + side, w1_ref[...],
                preferred_element_type=_f32) + b1_ref[...]
    a = jnp.where(a > 0, a, 0.01 * a)
    b = jnp.dot(x * side, w2_ref[...],
                preferred_element_type=_f32) + b2_ref[...]
    b = jnp.where(b > 0, b, 0.01 * b)
    e = a + b
    ego_ref[...] = e
    n = jnp.sqrt(jnp.sum(e * e, axis=1, keepdims=True))
    nrm_ref[...] = e / jnp.maximum(n, 1e-12)


def _layer_tc(x, side, w1, b1, w2, b2):
    n, din = x.shape
    dout = w1.shape[1]
    tile = 2000
    grid = (n // tile,)
    out = pl.pallas_call(
        _layer_body,
        grid=grid,
        in_specs=[
            pl.BlockSpec((tile, din), lambda i: (i, 0)),
            pl.BlockSpec((tile, din), lambda i: (i, 0)),
            pl.BlockSpec((din, dout), lambda i: (0, 0)),
            pl.BlockSpec((1, dout), lambda i: (0, 0)),
            pl.BlockSpec((din, dout), lambda i: (0, 0)),
            pl.BlockSpec((1, dout), lambda i: (0, 0)),
        ],
        out_specs=[
            pl.BlockSpec((tile, dout), lambda i: (i, 0)),
            pl.BlockSpec((tile, dout), lambda i: (i, 0)),
        ],
        out_shape=[
            jax.ShapeDtypeStruct((n, dout), _f32),
            jax.ShapeDtypeStruct((n, dout), _f32),
        ],
    )(x, side, w1, b1.reshape(1, dout), w2, b2.reshape(1, dout))
    return out


IDS_ROWS = 3 * BATCH // EPB


def _make_gather():
    mesh = plsc.VectorSubcoreMesh(
        core_axis_name="c", subcore_axis_name="s", num_cores=NC,
        num_subcores=NS)
    scratch = [
        pltpu.VMEM((IDS_ROWS, EPB), _i32),
        pltpu.VMEM((EPB, 128), _f32),
    ]

    @functools.partial(
        pl.kernel, mesh=mesh,
        out_type=jax.ShapeDtypeStruct((3, BATCH, 128), _f32),
        scratch_types=scratch,
        compiler_params=pltpu.CompilerParams(use_tc_tiling_on_sc=False),
    )
    def gather(ids_h, tab_h, out_h, ids_v, buf):
        c = lax.axis_index("c")
        s = lax.axis_index("s")
        w = s * NC + c
        pltpu.sync_copy(ids_h, ids_v)
        for t in range(3):
            r = t * NW + w
            pltpu.sync_copy(tab_h.at[ids_v.at[r]], buf)
            pltpu.sync_copy(buf, out_h.at[t, pl.ds(w * EPB, EPB)])

    return gather


_GATHER = _make_gather()


def _loss_body(upg_ref, out_ref):
    u = upg_ref[0]
    p = upg_ref[1]
    g = upg_ref[2]
    pos = jnp.sum(u * p, axis=1)
    neg = jnp.sum(u * g, axis=1)
    x = pos - neg
    softplus = jnp.maximum(-x, 0.0) + jnp.log1p(jnp.exp(-jnp.abs(x)))
    cf = jnp.mean(softplus)
    l2 = (jnp.mean(jnp.sum(u * u, axis=1)) +
          jnp.mean(jnp.sum(p * p, axis=1)) +
          jnp.mean(jnp.sum(g * g, axis=1))) * 0.5
    out_ref[...] = jnp.broadcast_to(cf + 1e-5 * l2, (1, 1))


def _loss_tc(upg):
    out = pl.pallas_call(
        _loss_body,
        out_shape=jax.ShapeDtypeStruct((1, 1), _f32),
    )(upg)
    return out[0, 0]


def kernel(user_ids, item_pos_ids, item_neg_ids, entity_user_embed,
           A_rows, A_cols, A_vals,
           W1_0, b1_0, W2_0, b2_0,
           W1_1, b1_1, W2_1, b2_1,
           W1_2, b1_2, W2_2, b2_2):
    pad = NNZ_PAD - NNZ
    rows3 = jnp.pad(A_rows.astype(_i32), (0, pad)).reshape(NW, S, EPB)
    cols3 = jnp.pad(A_cols.astype(_i32), (0, pad)).reshape(NW, S, EPB)
    vals3 = jnp.pad(A_vals, (0, pad)).reshape(NW, S, EPB)

    layers = [(W1_0, b1_0, W2_0, b2_0),
              (W1_1, b1_1, W2_1, b2_1),
              (W1_2, b1_2, W2_2, b2_2)]
    ego = entity_user_embed
    normed = []
    for (w1, b1, w2, b2) in layers:
        side = _spmm_sc(rows3, cols3, vals3, ego)
        ego, nrm = _layer_tc(ego, side, w1, b1, w2, b2)
        normed.append(nrm)

    all_e = jnp.concatenate([entity_user_embed] + normed, axis=1)

    ids = jnp.concatenate([
        user_ids.astype(_i32),
        item_pos_ids.astype(_i32),
        item_neg_ids.astype(_i32)]).reshape(IDS_ROWS, EPB)
    upg = _GATHER(ids, all_e)
    return _loss_tc(upg)
